# baseline (speedup 1.0000x reference)
import functools

import jax
import jax.numpy as jnp
from jax.experimental import pallas as pl
from jax.experimental.pallas import tpu as pltpu

N = 4096
D = 512
K = 31
BM = 256
TOL = 5e-7
MAX_ITERS = 40


def _fused_body(x_ref, w0_ref, w1_ref, out_ref, emb_ref):
    i = pl.program_id(0)

    @pl.when(i == 0)
    def _encode():
        h = x_ref[:] * w0_ref[:]
        h = jnp.maximum(h, 0.0)
        h = h * w1_ref[:]
        s = jnp.sum(h * h, axis=-1, keepdims=True)
        n = jnp.sqrt(s)
        emb_ref[:] = h / jnp.maximum(n, 1e-12)

    rows = emb_ref[pl.ds(i * BM, BM), :]
    a = jax.lax.dot_general(
        rows, emb_ref[:],
        dimension_numbers=(((1,), (1,)), ((), ())),
        preferred_element_type=jnp.float32,
    )

    def count_ge(thr):
        return jnp.sum(jnp.where(a >= thr, 1.0, 0.0), axis=1, keepdims=True)

    s1 = jnp.sum(a, axis=1, keepdims=True) * (1.0 / N)
    s2 = jnp.sum(a * a, axis=1, keepdims=True) * (1.0 / N)
    sig = jnp.sqrt(jnp.maximum(s2 - s1 * s1, 0.0))
    lo_c = s1 + 2.0 * sig
    hi_c = s1 + 5.0 * sig
    c_lo = count_ge(lo_c)
    c_hi = count_ge(hi_c)
    lo_ok = c_lo >= K
    hi_ok = c_hi < K
    lo = jnp.where(lo_ok, lo_c, -1.01)
    clo = jnp.where(lo_ok, c_lo, float(N))
    hi = jnp.where(hi_ok, hi_c, 1.01)
    chi = jnp.where(hi_ok, c_hi, 0.0)

    def cond(carry):
        it, lo, hi, clo, chi, last = carry
        return (it < MAX_ITERS) & (jnp.max(hi - lo) > TOL)

    def body(carry):
        it, lo, hi, clo, chi, last = carry
        frac = (clo - K) / jnp.maximum(clo - chi, 1.0)
        frac = jnp.clip(frac, 0.04, 0.96)
        frac = jnp.where(jnp.abs(last) >= 2.0, 0.5, frac)
        mid = lo + frac * (hi - lo)
        cnt = count_ge(mid)
        ge = cnt >= K
        side = jnp.where(ge, 1.0, -1.0)
        rep = side * last > 0.0
        streak = jnp.where(rep, jnp.abs(last) + 1.0, 1.0)
        return (it + 1,
                jnp.where(ge, mid, lo),
                jnp.where(ge, hi, mid),
                jnp.where(ge, cnt, clo),
                jnp.where(ge, chi, cnt),
                streak * side)

    zero = jnp.zeros((BM, 1), jnp.float32)
    _, lo, hi, clo, chi, _ = jax.lax.while_loop(
        cond, body, (0, lo, hi, clo, chi, zero))
    out_ref[:] = jnp.where((a >= lo) & (a > 0.0), a, 0.0)


@jax.jit
def kernel(x, w0, w1):
    return pl.pallas_call(
        _fused_body,
        grid=(N // BM,),
        in_specs=[
            pl.BlockSpec((N, D), lambda i: (0, 0)),
            pl.BlockSpec((1, D), lambda i: (0, 0)),
            pl.BlockSpec((1, D), lambda i: (0, 0)),
        ],
        out_specs=pl.BlockSpec((BM, N), lambda i: (i, 0)),
        out_shape=jax.ShapeDtypeStruct((N, N), jnp.float32),
        scratch_shapes=[pltpu.VMEM((N, D), jnp.float32)],
    )(x, w0.reshape(1, D), w1.reshape(1, D))


# speedup vs baseline: 1.6899x; 1.6899x over previous
import jax
import jax.numpy as jnp
from jax.experimental import pallas as pl
from jax.experimental.pallas import tpu as pltpu

N = 4096
D = 512
K = 31
BM = 256
BISECT_ITERS = 26


def _fused_body(x_ref, w0_ref, w1_ref, out_ref, emb_ref):
    i = pl.program_id(0)

    @pl.when(i == 0)
    def _encode():
        h = x_ref[:] * w0_ref[:]
        h = jnp.maximum(h, 0.0)
        h = h * w1_ref[:]
        s = jnp.sum(h * h, axis=-1, keepdims=True)
        n = jnp.sqrt(s)
        emb_ref[:] = h / jnp.maximum(n, 1e-12)

    rows = emb_ref[pl.ds(i * BM, BM), :]
    a = jax.lax.dot_general(
        rows, emb_ref[:],
        dimension_numbers=(((1,), (1,)), ((), ())),
        preferred_element_type=jnp.float32,
    )

    def body(_, carry):
        lo, hi, clo = carry
        mid = (lo + hi) * 0.5
        cnt = jnp.sum(jnp.where(a >= mid, 1.0, 0.0), axis=1, keepdims=True)
        ge = cnt >= K
        return (jnp.where(ge, mid, lo),
                jnp.where(ge, hi, mid),
                jnp.where(ge, cnt, clo))

    lo0 = jnp.full((BM, 1), -1.01, jnp.float32)
    hi0 = jnp.full((BM, 1), 1.01, jnp.float32)
    clo0 = jnp.full((BM, 1), float(N), jnp.float32)
    lo, _, clo = jax.lax.fori_loop(0, BISECT_ITERS, body, (lo0, hi0, clo0))
    t = jnp.min(jnp.where(a >= lo, a, 2.0), axis=1, keepdims=True)
    t = jnp.where(clo == K, t, lo)
    out_ref[:] = jnp.where((a >= t) & (a > 0.0), a, 0.0)


@jax.jit
def kernel(x, w0, w1):
    return pl.pallas_call(
        _fused_body,
        grid=(N // BM,),
        in_specs=[
            pl.BlockSpec((N, D), lambda i: (0, 0)),
            pl.BlockSpec((1, D), lambda i: (0, 0)),
            pl.BlockSpec((1, D), lambda i: (0, 0)),
        ],
        out_specs=pl.BlockSpec((BM, N), lambda i: (i, 0)),
        out_shape=jax.ShapeDtypeStruct((N, N), jnp.float32),
        scratch_shapes=[pltpu.VMEM((N, D), jnp.float32)],
    )(x, w0.reshape(1, D), w1.reshape(1, D))


# speedup vs baseline: 1.8244x; 1.0796x over previous
import jax
import jax.numpy as jnp
from jax.experimental import pallas as pl
from jax.experimental.pallas import tpu as pltpu

N = 4096
D = 512
K = 31
BM = 512
BISECT_ITERS = 26


def _fused_body(x_ref, w0_ref, w1_ref, out_ref, emb_ref):
    i = pl.program_id(0)

    @pl.when(i == 0)
    def _encode():
        h = x_ref[:] * w0_ref[:]
        h = jnp.maximum(h, 0.0)
        h = h * w1_ref[:]
        s = jnp.sum(h * h, axis=-1, keepdims=True)
        n = jnp.sqrt(s)
        emb_ref[:] = h / jnp.maximum(n, 1e-12)

    rows = emb_ref[pl.ds(i * BM, BM), :]
    a = jax.lax.dot_general(
        rows, emb_ref[:],
        dimension_numbers=(((1,), (1,)), ((), ())),
        preferred_element_type=jnp.float32,
    )

    def body(_, carry):
        lo, hi, clo = carry
        mid = (lo + hi) * 0.5
        cnt = jnp.sum(jnp.where(a >= mid, 1.0, 0.0), axis=1, keepdims=True)
        ge = cnt >= K
        return (jnp.where(ge, mid, lo),
                jnp.where(ge, hi, mid),
                jnp.where(ge, cnt, clo))

    lo0 = jnp.full((BM, 1), -1.01, jnp.float32)
    hi0 = jnp.full((BM, 1), 1.01, jnp.float32)
    clo0 = jnp.full((BM, 1), float(N), jnp.float32)
    lo, _, clo = jax.lax.fori_loop(0, BISECT_ITERS, body, (lo0, hi0, clo0))
    t = jnp.min(jnp.where(a >= lo, a, 2.0), axis=1, keepdims=True)
    t = jnp.where(clo == K, t, lo)
    out_ref[:] = jnp.where((a >= t) & (a > 0.0), a, 0.0)


@jax.jit
def kernel(x, w0, w1):
    return pl.pallas_call(
        _fused_body,
        grid=(N // BM,),
        in_specs=[
            pl.BlockSpec((N, D), lambda i: (0, 0)),
            pl.BlockSpec((1, D), lambda i: (0, 0)),
            pl.BlockSpec((1, D), lambda i: (0, 0)),
        ],
        out_specs=pl.BlockSpec((BM, N), lambda i: (i, 0)),
        out_shape=jax.ShapeDtypeStruct((N, N), jnp.float32),
        scratch_shapes=[pltpu.VMEM((N, D), jnp.float32)],
    )(x, w0.reshape(1, D), w1.reshape(1, D))
